# Initial kernel scaffold; baseline (speedup 1.0000x reference)
import jax
import jax.numpy as jnp
from jax.experimental import pallas as pl


def kernel(params, template, reference_img):
    raise NotImplementedError("write your pallas kernel here")


# speedup vs baseline: 3.4958x; 3.4958x over previous
import functools

import jax
import jax.numpy as jnp
from jax import lax
from jax.experimental import pallas as pl
from jax.experimental.pallas import tpu as pltpu
from jax.experimental.pallas import tpu_sc as plsc

XS = 512
YS = 512
NC = 2
NS = 16
L = 16
NW = NC * NS

TH = 64
TW = 128
GR = XS // TH
GC = YS // TW

R = 160
C = 384

GROUPS = TH * (TW // L)


def _floor_i32(x):
  xi = x.astype(jnp.int32)
  return jnp.where(xi.astype(jnp.float32) > x, xi - 1, xi)


def _bf16r(v):
  b = plsc.bitcast(v, jnp.int32)
  r = (b + 0x7FFF + ((b >> 16) & 1)) & jnp.int32(-65536)
  return plsc.bitcast(r, jnp.float32)


def _sc_body(tmpl_hbm, ref_hbm, coef_hbm, out_hbm, win_v, refs_v, coef_v, part_v):
  cid = lax.axis_index("c")
  sid = lax.axis_index("s")
  wid = sid * NC + cid
  tr = wid // GC
  tc = wid % GC
  i0 = tr * TH
  j0 = tc * TW

  pltpu.sync_copy(coef_hbm, coef_v)
  m00 = jnp.max(coef_v[0, :])
  m01 = jnp.max(coef_v[1, :])
  m02 = jnp.max(coef_v[2, :])
  m10 = jnp.max(coef_v[3, :])
  m11 = jnp.max(coef_v[4, :])
  m12 = jnp.max(coef_v[5, :])

  fi0 = i0.astype(jnp.float32)
  fi1 = (i0 + TH - 1).astype(jnp.float32)
  fj0 = j0.astype(jnp.float32)
  fj1 = (j0 + TW - 1).astype(jnp.float32)
  xmin = jnp.minimum(jnp.minimum(m00 * fi0 + m01 * fj0, m00 * fi0 + m01 * fj1),
                     jnp.minimum(m00 * fi1 + m01 * fj0, m00 * fi1 + m01 * fj1)) + m02
  ymin = jnp.minimum(jnp.minimum(m10 * fi0 + m11 * fj0, m10 * fi0 + m11 * fj1),
                     jnp.minimum(m10 * fi1 + m11 * fj0, m10 * fi1 + m11 * fj1)) + m12
  row0 = pl.multiple_of(jnp.clip(_floor_i32(xmin) - 4, 0, XS - R) & ~7, 8)
  col0 = pl.multiple_of(jnp.clip(_floor_i32(ymin) - 4, 0, YS - C) & ~127, 128)

  pltpu.sync_copy(tmpl_hbm.at[pl.ds(row0, R), pl.ds(col0, C)], win_v)
  pltpu.sync_copy(ref_hbm.at[pl.ds(i0, TH), pl.ds(j0, TW)], refs_v)

  lanes = lax.iota(jnp.int32, L)
  one = jnp.full((L,), 1.0, jnp.float32)

  def step(t, acc):
    i = t // (TW // L)
    g = t % (TW // L)
    riv = _bf16r(jnp.broadcast_to((i0 + i).astype(jnp.float32), (L,)))
    rjv = _bf16r((j0 + g * L + lanes).astype(jnp.float32))
    x = (riv * m00 + rjv * m01) + m02
    y = (riv * m10 + rjv * m11) + m12
    x0 = jnp.clip(_floor_i32(x), 1, XS - 2)
    y0 = jnp.clip(_floor_i32(y), 1, YS - 2)
    wx = x - x0.astype(jnp.float32)
    wy = y - y0.astype(jnp.float32)
    lx = jnp.clip(x0 - row0, 0, R - 2)
    ly = jnp.clip(y0 - col0, 0, C - 2)
    lx1 = lx + 1
    ly1 = ly + 1
    v00 = plsc.load_gather(win_v, [lx, ly])
    v01 = plsc.load_gather(win_v, [lx, ly1])
    v10 = plsc.load_gather(win_v, [lx1, ly])
    v11 = plsc.load_gather(win_v, [lx1, ly1])
    top = (one - wy) * v00 + wy * v01
    bot = (one - wy) * v10 + wy * v11
    val = (one - wx) * top + wx * bot
    rv = refs_v[i, pl.ds(g * L, L)]
    return acc + val * rv

  acc = lax.fori_loop(0, GROUPS, step, jnp.zeros((L,), jnp.float32))
  part_v[...] = acc
  pltpu.sync_copy(part_v, out_hbm.at[wid])


_sc_warp = pl.kernel(
    _sc_body,
    out_type=jax.ShapeDtypeStruct((NW, L), jnp.float32),
    mesh=plsc.VectorSubcoreMesh(core_axis_name="c", subcore_axis_name="s",
                                num_cores=NC, num_subcores=NS),
    scratch_types=[
        pltpu.VMEM((R, C), jnp.float32),
        pltpu.VMEM((TH, TW), jnp.float32),
        pltpu.VMEM((8, L), jnp.float32),
        pltpu.VMEM((L,), jnp.float32),
    ],
    compiler_params=pltpu.CompilerParams(needs_layout_passes=False),
)


def _finish_body(p_ref, o_ref):
  s = jnp.sum(p_ref[...])
  o_ref[...] = jnp.broadcast_to(s * s, (1, 1))


_finish = pl.pallas_call(
    _finish_body,
    out_shape=jax.ShapeDtypeStruct((1, 1), jnp.float32),
)


def kernel(params, template, reference_img):
  if params.ndim > 1:
    params = params[0]
  angle, tx, ty = params[0], params[1], params[2]
  cx = float(YS // 2)
  cy = float(XS // 2)
  rot = jnp.deg2rad(angle)
  c = jnp.cos(rot)
  s = jnp.sin(rot)
  m02 = c * (-tx - cx) - s * (-ty - cy) + cx
  m12 = s * (-tx - cx) + c * (-ty - cy) + cy
  coefs = jnp.stack([c, -s, m02, s, c, m12,
                     jnp.float32(0.0), jnp.float32(0.0)]).astype(jnp.float32)
  coefs = coefs.astype(jnp.bfloat16).astype(jnp.float32)
  coefs = jnp.broadcast_to(coefs[:, None], (8, L))
  partials = _sc_warp(template, reference_img, coefs)
  return _finish(partials)


# speedup vs baseline: 3.6331x; 1.0393x over previous
import functools

import jax
import jax.numpy as jnp
from jax import lax
from jax.experimental import pallas as pl
from jax.experimental.pallas import tpu as pltpu
from jax.experimental.pallas import tpu_sc as plsc

XS = 512
YS = 512
NC = 2
NS = 16
L = 16
NW = NC * NS

TH = 64
TW = 128
GR = XS // TH
GC = YS // TW

R = 160
C = 384

GROUPS = TH * (TW // L)


def _floor_i32(x):
  xi = x.astype(jnp.int32)
  return jnp.where(xi.astype(jnp.float32) > x, xi - 1, xi)


def _bf16r(v):
  b = plsc.bitcast(v, jnp.int32)
  r = (b + 0x7FFF + ((b >> 16) & 1)) & jnp.int32(-65536)
  return plsc.bitcast(r, jnp.float32)


def _sc_body(tmpl_hbm, ref_hbm, coef_hbm, out_hbm, win_v, refs_v, coef_v, part_v,
             dma_sem):
  cid = lax.axis_index("c")
  sid = lax.axis_index("s")
  wid = sid * NC + cid
  tr = wid // GC
  tc = wid % GC
  i0 = tr * TH
  j0 = tc * TW

  pltpu.sync_copy(coef_hbm, coef_v)
  m00 = jnp.max(coef_v[0, :])
  m01 = jnp.max(coef_v[1, :])
  m02 = jnp.max(coef_v[2, :])
  m10 = jnp.max(coef_v[3, :])
  m11 = jnp.max(coef_v[4, :])
  m12 = jnp.max(coef_v[5, :])

  fi0 = i0.astype(jnp.float32)
  fi1 = (i0 + TH - 1).astype(jnp.float32)
  fj0 = j0.astype(jnp.float32)
  fj1 = (j0 + TW - 1).astype(jnp.float32)
  xmin = jnp.minimum(jnp.minimum(m00 * fi0 + m01 * fj0, m00 * fi0 + m01 * fj1),
                     jnp.minimum(m00 * fi1 + m01 * fj0, m00 * fi1 + m01 * fj1)) + m02
  ymin = jnp.minimum(jnp.minimum(m10 * fi0 + m11 * fj0, m10 * fi0 + m11 * fj1),
                     jnp.minimum(m10 * fi1 + m11 * fj0, m10 * fi1 + m11 * fj1)) + m12
  row0 = pl.multiple_of(jnp.clip(_floor_i32(xmin) - 4, 0, XS - R) & ~7, 8)
  col0 = pl.multiple_of(jnp.clip(_floor_i32(ymin) - 4, 0, YS - C) & ~127, 128)

  wcopy = pltpu.async_copy(tmpl_hbm.at[pl.ds(row0, R), pl.ds(col0, C)], win_v,
                           dma_sem)
  pltpu.sync_copy(ref_hbm.at[pl.ds(i0, TH), pl.ds(j0, TW)], refs_v)
  wcopy.wait()

  lanes = lax.iota(jnp.int32, L)
  one = jnp.full((L,), 1.0, jnp.float32)
  NG = TW // L
  rjx = [_bf16r((j0 + g * L + lanes).astype(jnp.float32)) * m01 for g in range(NG)]
  rjy = [_bf16r((j0 + g * L + lanes).astype(jnp.float32)) * m11 for g in range(NG)]

  @plsc.parallel_loop(0, TH, carry=jnp.zeros((L,), jnp.float32))
  def row_loop(i, acc):
    riv = _bf16r(jnp.broadcast_to((i0 + i).astype(jnp.float32), (L,)))
    rowx = riv * m00 + m02
    rowy = riv * m10 + m12
    for g in range(NG):
      x = rowx + rjx[g]
      y = rowy + rjy[g]
      x0 = jnp.clip(_floor_i32(x), 1, XS - 2)
      y0 = jnp.clip(_floor_i32(y), 1, YS - 2)
      wx = x - x0.astype(jnp.float32)
      wy = y - y0.astype(jnp.float32)
      lx = x0 - row0
      ly = y0 - col0
      lx1 = lx + 1
      ly1 = ly + 1
      v00 = plsc.load_gather(win_v, [lx, ly])
      v01 = plsc.load_gather(win_v, [lx, ly1])
      v10 = plsc.load_gather(win_v, [lx1, ly])
      v11 = plsc.load_gather(win_v, [lx1, ly1])
      top = (one - wy) * v00 + wy * v01
      bot = (one - wy) * v10 + wy * v11
      val = (one - wx) * top + wx * bot
      rv = refs_v[i, pl.ds(g * L, L)]
      acc = acc + val * rv
    return acc

  part_v[...] = row_loop
  pltpu.sync_copy(part_v, out_hbm.at[wid])


_sc_warp = pl.kernel(
    _sc_body,
    out_type=jax.ShapeDtypeStruct((NW, L), jnp.float32),
    mesh=plsc.VectorSubcoreMesh(core_axis_name="c", subcore_axis_name="s",
                                num_cores=NC, num_subcores=NS),
    scratch_types=[
        pltpu.VMEM((R, C), jnp.float32),
        pltpu.VMEM((TH, TW), jnp.float32),
        pltpu.VMEM((8, L), jnp.float32),
        pltpu.VMEM((L,), jnp.float32),
        pltpu.SemaphoreType.DMA,
    ],
    compiler_params=pltpu.CompilerParams(needs_layout_passes=False),
)


def _finish_body(p_ref, o_ref):
  s = jnp.sum(p_ref[...])
  o_ref[...] = jnp.broadcast_to(s * s, (1, 1))


_finish = pl.pallas_call(
    _finish_body,
    out_shape=jax.ShapeDtypeStruct((1, 1), jnp.float32),
)


def kernel(params, template, reference_img):
  if params.ndim > 1:
    params = params[0]
  angle, tx, ty = params[0], params[1], params[2]
  cx = float(YS // 2)
  cy = float(XS // 2)
  rot = jnp.deg2rad(angle)
  c = jnp.cos(rot)
  s = jnp.sin(rot)
  m02 = c * (-tx - cx) - s * (-ty - cy) + cx
  m12 = s * (-tx - cx) + c * (-ty - cy) + cy
  coefs = jnp.stack([c, -s, m02, s, c, m12,
                     jnp.float32(0.0), jnp.float32(0.0)]).astype(jnp.float32)
  coefs = coefs.astype(jnp.bfloat16).astype(jnp.float32)
  coefs = jnp.broadcast_to(coefs[:, None], (8, L))
  partials = _sc_warp(template, reference_img, coefs)
  return _finish(partials)
